# baseline (speedup 1.0000x reference)
import functools

import jax
import jax.numpy as jnp
from jax import lax
from jax.experimental import pallas as pl
from jax.experimental.pallas import tpu as pltpu
from jax.experimental.pallas import tpu_sc as plsc

N = 10000
NE = 320000
D = 128
H = 8
DH = D // H
SCALE = 1.0 / (DH ** 0.5)

N_PAD = 10240
FLAT = N_PAD * N_PAD
BC = 256
BN = 1024
NI = N_PAD // BC
NJ = N_PAD // BN

NC = 2
NS = 16
NW = NC * NS
E_PER_W = NE // NW
EBATCH = 2000
NBATCH = E_PER_W // EBATCH
VPB = EBATCH // 16
CHUNKS = 160

PIECE_ROWS = [256, 1536, 1792, 1792, 1792, 1792, 1024, 256]
PIECE_LO = [sum(PIECE_ROWS[:p]) for p in range(len(PIECE_ROWS))]
NPIECE = len(PIECE_ROWS)


def _make_sc_body(lo, rows):
    hi = lo + rows
    dump = rows * N_PAD

    def body(src_hbm, dst_hbm, mask_hbm, idx1d, idx2d, ones_v, sbuf, dbuf,
             ssem):
        cid = lax.axis_index("c")
        sid = lax.axis_index("s")
        wid = cid * NS + sid
        lanes = lax.iota(jnp.int32, 16)
        for g in range(8):
            ones_v[pl.ds(g * 16, 16)] = jnp.ones((16,), jnp.float32)

        ebase = wid * E_PER_W

        def batch_body(b, ptr):
            pltpu.sync_copy(src_hbm.at[pl.ds(ebase + b * EBATCH, EBATCH)],
                            sbuf)
            pltpu.sync_copy(dst_hbm.at[pl.ds(ebase + b * EBATCH, EBATCH)],
                            dbuf)

            def vec(i, ptr):
                s16 = sbuf[pl.ds(i * 16, 16)]
                d16 = dbuf[pl.ds(i * 16, 16)]
                nsl = s16 != d16
                in1 = nsl & (s16 >= lo) & (s16 < hi)
                in2 = nsl & (d16 >= lo) & (d16 < hi)
                f1 = (s16 - lo) * N_PAD + d16
                f2 = (d16 - lo) * N_PAD + s16
                cs1 = plsc.cumsum(in1.astype(jnp.int32))
                plsc.store_scatter(idx1d, [ptr + cs1 - 1], f1, mask=in1)
                ptr = ptr + cs1[15]
                cs2 = plsc.cumsum(in2.astype(jnp.int32))
                plsc.store_scatter(idx1d, [ptr + cs2 - 1], f2, mask=in2)
                ptr = ptr + cs2[15]
                return ptr
            return lax.fori_loop(0, VPB, vec, ptr)
        ptr = lax.fori_loop(0, NBATCH, batch_body, jnp.int32(0))

        def pad(t, c):
            idx1d[pl.ds(ptr + t * 16, 16)] = dump + lanes
            return c
        lax.fori_loop(0, 8, pad, 0)
        nchunks = (ptr + 127) // 128

        def cp(i, c):
            idx2d[i // 8, pl.ds((i % 8) * 16, 16)] = idx1d[pl.ds(i * 16, 16)]
            return c
        lax.fori_loop(0, nchunks * 8, cp, 0)

        def fire(t, c):
            pltpu.async_copy(ones_v, mask_hbm.at[idx2d.at[t]], ssem)
            return c
        lax.fori_loop(0, nchunks, fire, 0)

        def drain(t, c):
            pltpu.make_async_copy(ones_v, mask_hbm.at[idx2d.at[0]],
                                  ssem).wait()
            return c
        lax.fori_loop(0, nchunks, drain, 0)
    return body


def _make_sc_scatter(lo, rows):
    return pl.kernel(
        _make_sc_body(lo, rows),
        out_type=(),
        compiler_params=pltpu.CompilerParams(needs_layout_passes=False),
        mesh=plsc.VectorSubcoreMesh(core_axis_name="c", subcore_axis_name="s"),
        scratch_types=[
            pltpu.VMEM((20608,), jnp.int32),
            pltpu.VMEM((CHUNKS, 128), jnp.int32),
            pltpu.VMEM((128,), jnp.float32),
            pltpu.VMEM((EBATCH,), jnp.int32),
            pltpu.VMEM((EBATCH,), jnp.int32),
            pltpu.SemaphoreType.DMA,
        ],
    )


_SC_SCATTERS = [_make_sc_scatter(PIECE_LO[p], PIECE_ROWS[p])
                for p in range(NPIECE)]


VW = 32


def _proj_body(emb_ref, w_ref, b_ref, q_ref, k_ref, v_ref):
    bp = emb_ref.shape[0]
    y = lax.dot_general(emb_ref[...], w_ref[...], (((1,), (1,)), ((), ())),
                        preferred_element_type=jnp.float32) + b_ref[...]
    q_ref[...] = y[:, :D] * SCALE
    k_ref[...] = y[:, D:2 * D]
    col = lax.broadcasted_iota(jnp.int32, (bp, VW - DH), 1)
    tail = jnp.where(col == 0, 1.0, 0.0)
    for h in range(H):
        v_ref[:, VW * h:VW * h + DH] = y[:, 2 * D + DH * h:2 * D + DH * (h + 1)]
        v_ref[:, VW * h + DH:VW * (h + 1)] = tail


def _proj(emb_pad, w_all, b_all):
    bp = 512
    return pl.pallas_call(
        _proj_body,
        grid=(N_PAD // bp,),
        in_specs=[
            pl.BlockSpec((bp, D), lambda i: (i, 0)),
            pl.BlockSpec((3 * D, D), lambda i: (0, 0)),
            pl.BlockSpec((1, 3 * D), lambda i: (0, 0)),
        ],
        out_specs=[
            pl.BlockSpec((bp, D), lambda i: (i, 0)),
            pl.BlockSpec((bp, D), lambda i: (i, 0)),
            pl.BlockSpec((bp, H * VW), lambda i: (i, 0)),
        ],
        out_shape=[jax.ShapeDtypeStruct((N_PAD, D), jnp.float32),
                   jax.ShapeDtypeStruct((N_PAD, D), jnp.float32),
                   jax.ShapeDtypeStruct((N_PAD, H * VW), jnp.float32)],
    )(emb_pad, w_all, b_all)


def _flash_body(q_ref, k_ref, v_ref, mask_ref, emb_ref, wo_ref, bo_ref,
                wl_ref, bl_ref, lnw_ref, lnb_ref, out_ref,
                acc, mscr):
    j = pl.program_id(1)

    @pl.when(j == 0)
    def _():
        acc[...] = jnp.zeros((BC, H * VW), jnp.float32)
        mscr[...] = jnp.full((BC, H), -jnp.inf, jnp.float32)

    neg = -jnp.inf
    bias = jnp.where(mask_ref[...] > 0.0, 0.0, neg)
    for h in range(H):
        qh = q_ref[:, h * DH:(h + 1) * DH]
        kh = k_ref[pl.ds(j * BN, BN), h * DH:(h + 1) * DH]
        s = lax.dot_general(qh, kh, (((1,), (1,)), ((), ())),
                            preferred_element_type=jnp.float32) + bias
        mo = mscr[:, h:h + 1]
        mn = jnp.maximum(mo, jnp.max(s, axis=1, keepdims=True))
        msafe = jnp.where(mn > neg, mn, 0.0)
        p = jnp.exp(s - msafe)
        alpha = jnp.where(mn > neg, jnp.exp(mo - mn), 0.0)
        vh = v_ref[pl.ds(j * BN, BN), VW * h:VW * (h + 1)]
        pv = lax.dot_general(p, vh, (((1,), (0,)), ((), ())),
                             preferred_element_type=jnp.float32)
        acc[:, VW * h:VW * (h + 1)] = acc[:, VW * h:VW * (h + 1)] * alpha + pv
        mscr[:, h:h + 1] = mn

    @pl.when(j == NJ - 1)
    def _():
        parts = [acc[:, VW * h:VW * h + DH] /
                 acc[:, VW * h + DH:VW * h + DH + 1] for h in range(H)]
        ctx = jnp.concatenate(parts, axis=1)
        ctxp = lax.dot_general(ctx, wo_ref[...], (((1,), (1,)), ((), ())),
                               preferred_element_type=jnp.float32) + bo_ref[...]
        has = acc[:, DH:DH + 1] > 0.0
        c2 = jnp.where(has, ctxp, emb_ref[...])
        h1 = lax.dot_general(c2, wl_ref[...], (((1,), (1,)), ((), ())),
                             preferred_element_type=jnp.float32) + bl_ref[...]
        mu = jnp.mean(h1, axis=1, keepdims=True)
        var = jnp.mean((h1 - mu) ** 2, axis=1, keepdims=True)
        hn = (h1 - mu) / jnp.sqrt(var + 1e-5) * lnw_ref[...] + lnb_ref[...]
        out_ref[...] = 0.5 * hn * (1.0 + lax.erf(hn * (2.0 ** -0.5)))


def _flash_piece(p, q, k, v, maskp, emb_pad, wo, bo, wl, bl, lnw, lnb):
    i0 = PIECE_LO[p] // BC
    ni_p = PIECE_ROWS[p] // BC

    def cmap(i, j, i0=i0):
        return (i + i0, 0)

    return pl.pallas_call(
        _flash_body,
        grid=(ni_p, NJ),
        in_specs=[
            pl.BlockSpec((BC, D), cmap),
            pl.BlockSpec((N_PAD, D), lambda i, j: (0, 0)),
            pl.BlockSpec((N_PAD, H * VW), lambda i, j: (0, 0)),
            pl.BlockSpec((BC, BN), lambda i, j: (i, j)),
            pl.BlockSpec((BC, D), cmap),
            pl.BlockSpec((D, D), lambda i, j: (0, 0)),
            pl.BlockSpec((1, D), lambda i, j: (0, 0)),
            pl.BlockSpec((D, D), lambda i, j: (0, 0)),
            pl.BlockSpec((1, D), lambda i, j: (0, 0)),
            pl.BlockSpec((1, D), lambda i, j: (0, 0)),
            pl.BlockSpec((1, D), lambda i, j: (0, 0)),
        ],
        out_specs=pl.BlockSpec((BC, D), lambda i, j: (i, 0)),
        out_shape=jax.ShapeDtypeStruct((PIECE_ROWS[p], D), jnp.float32),
        scratch_shapes=[
            pltpu.VMEM((BC, H * VW), jnp.float32),
            pltpu.VMEM((BC, H), jnp.float32),
        ],
        compiler_params=pltpu.CompilerParams(
            dimension_semantics=("arbitrary", "arbitrary")),
    )(q, k, v, maskp, emb_pad, wo, bo, wl, bl, lnw, lnb)


def kernel(embeddings, edge_index, in_proj_w, in_proj_b, out_proj_w,
           out_proj_b, lin_w, lin_b, ln_w, ln_b):
    emb_pad = jnp.zeros((N_PAD, D), jnp.float32).at[:N].set(embeddings)
    src = edge_index[0].astype(jnp.int32)
    dst = edge_index[1].astype(jnp.int32)

    q, k, v = _proj(emb_pad, in_proj_w, in_proj_b.reshape(1, 3 * D))

    masks = []
    for p in range(NPIECE):
        rows = PIECE_ROWS[p]
        mref = jax.new_ref(jnp.zeros(((rows + 8) * N_PAD,), jnp.float32))
        _SC_SCATTERS[p](src, dst, mref)
        masks.append(mref[...].reshape(rows + 8, N_PAD))

    outs = []
    for p in range(NPIECE):
        outs.append(_flash_piece(
            p, q, k, v, masks[p], emb_pad,
            out_proj_w, out_proj_b.reshape(1, D),
            lin_w, lin_b.reshape(1, D),
            ln_w.reshape(1, D), ln_b.reshape(1, D)))
    return jnp.concatenate(outs, axis=0)[:N]


# speedup vs baseline: 1.0096x; 1.0096x over previous
import functools

import jax
import jax.numpy as jnp
from jax import lax
from jax.experimental import pallas as pl
from jax.experimental.pallas import tpu as pltpu
from jax.experimental.pallas import tpu_sc as plsc

N = 10000
NE = 320000
D = 128
H = 8
DH = D // H
SCALE = 1.0 / (DH ** 0.5)

N_PAD = 10240
FLAT = N_PAD * N_PAD
BC = 256
BN = 1024
NI = N_PAD // BC
NJ = N_PAD // BN

NC = 2
NS = 16
NW = NC * NS
E_PER_W = NE // NW
EBATCH = 2000
NBATCH = E_PER_W // EBATCH
VPB = EBATCH // 16
CHUNKS = 160

PIECE_ROWS = [1280] * 8
PIECE_LO = [sum(PIECE_ROWS[:p]) for p in range(len(PIECE_ROWS))]
NPIECE = len(PIECE_ROWS)


def _make_sc_body(lo, rows):
    hi = lo + rows
    dump = rows * N_PAD

    def body(src_hbm, dst_hbm, mask_hbm, idx1d, idx2d, ones_v, sbuf, dbuf,
             ssem):
        cid = lax.axis_index("c")
        sid = lax.axis_index("s")
        wid = cid * NS + sid
        lanes = lax.iota(jnp.int32, 16)
        for g in range(8):
            ones_v[pl.ds(g * 16, 16)] = jnp.ones((16,), jnp.float32)

        ebase = wid * E_PER_W

        def batch_body(b, ptr):
            pltpu.sync_copy(src_hbm.at[pl.ds(ebase + b * EBATCH, EBATCH)],
                            sbuf)
            pltpu.sync_copy(dst_hbm.at[pl.ds(ebase + b * EBATCH, EBATCH)],
                            dbuf)

            def vec(i, ptr):
                s16 = sbuf[pl.ds(i * 16, 16)]
                d16 = dbuf[pl.ds(i * 16, 16)]
                nsl = s16 != d16
                in1 = nsl & (s16 >= lo) & (s16 < hi)
                in2 = nsl & (d16 >= lo) & (d16 < hi)
                f1 = (s16 - lo) * N_PAD + d16
                f2 = (d16 - lo) * N_PAD + s16
                cs1 = plsc.cumsum(in1.astype(jnp.int32))
                plsc.store_scatter(idx1d, [ptr + cs1 - 1], f1, mask=in1)
                ptr = ptr + cs1[15]
                cs2 = plsc.cumsum(in2.astype(jnp.int32))
                plsc.store_scatter(idx1d, [ptr + cs2 - 1], f2, mask=in2)
                ptr = ptr + cs2[15]
                return ptr
            return lax.fori_loop(0, VPB, vec, ptr)
        ptr = lax.fori_loop(0, NBATCH, batch_body, jnp.int32(0))

        def pad(t, c):
            idx1d[pl.ds(ptr + t * 16, 16)] = dump + lanes
            return c
        lax.fori_loop(0, 8, pad, 0)
        nchunks = (ptr + 127) // 128

        def cp(i, c):
            idx2d[i // 8, pl.ds((i % 8) * 16, 16)] = idx1d[pl.ds(i * 16, 16)]
            return c
        lax.fori_loop(0, nchunks * 8, cp, 0)

        def fire(t, c):
            pltpu.async_copy(ones_v, mask_hbm.at[idx2d.at[t]], ssem)
            return c
        lax.fori_loop(0, nchunks, fire, 0)

        def drain(t, c):
            pltpu.make_async_copy(ones_v, mask_hbm.at[idx2d.at[0]],
                                  ssem).wait()
            return c
        lax.fori_loop(0, nchunks, drain, 0)
    return body


def _make_sc_scatter(lo, rows):
    return pl.kernel(
        _make_sc_body(lo, rows),
        out_type=(),
        compiler_params=pltpu.CompilerParams(needs_layout_passes=False),
        mesh=plsc.VectorSubcoreMesh(core_axis_name="c", subcore_axis_name="s"),
        scratch_types=[
            pltpu.VMEM((20608,), jnp.int32),
            pltpu.VMEM((CHUNKS, 128), jnp.int32),
            pltpu.VMEM((128,), jnp.float32),
            pltpu.VMEM((EBATCH,), jnp.int32),
            pltpu.VMEM((EBATCH,), jnp.int32),
            pltpu.SemaphoreType.DMA,
        ],
    )


_SC_SCATTERS = [_make_sc_scatter(PIECE_LO[p], PIECE_ROWS[p])
                for p in range(NPIECE)]


VW = 32


def _proj_body(emb_ref, w_ref, b_ref, q_ref, k_ref, v_ref):
    bp = emb_ref.shape[0]
    y = lax.dot_general(emb_ref[...], w_ref[...], (((1,), (1,)), ((), ())),
                        preferred_element_type=jnp.float32) + b_ref[...]
    q_ref[...] = y[:, :D] * SCALE
    k_ref[...] = y[:, D:2 * D]
    col = lax.broadcasted_iota(jnp.int32, (bp, VW - DH), 1)
    tail = jnp.where(col == 0, 1.0, 0.0)
    for h in range(H):
        v_ref[:, VW * h:VW * h + DH] = y[:, 2 * D + DH * h:2 * D + DH * (h + 1)]
        v_ref[:, VW * h + DH:VW * (h + 1)] = tail


def _proj(emb_pad, w_all, b_all):
    bp = 512
    return pl.pallas_call(
        _proj_body,
        grid=(N_PAD // bp,),
        in_specs=[
            pl.BlockSpec((bp, D), lambda i: (i, 0)),
            pl.BlockSpec((3 * D, D), lambda i: (0, 0)),
            pl.BlockSpec((1, 3 * D), lambda i: (0, 0)),
        ],
        out_specs=[
            pl.BlockSpec((bp, D), lambda i: (i, 0)),
            pl.BlockSpec((bp, D), lambda i: (i, 0)),
            pl.BlockSpec((bp, H * VW), lambda i: (i, 0)),
        ],
        out_shape=[jax.ShapeDtypeStruct((N_PAD, D), jnp.float32),
                   jax.ShapeDtypeStruct((N_PAD, D), jnp.float32),
                   jax.ShapeDtypeStruct((N_PAD, H * VW), jnp.float32)],
    )(emb_pad, w_all, b_all)


def _flash_body(q_ref, k_ref, v_ref, mask_ref, emb_ref, wo_ref, bo_ref,
                wl_ref, bl_ref, lnw_ref, lnb_ref, out_ref,
                acc, mscr):
    j = pl.program_id(1)

    @pl.when(j == 0)
    def _():
        acc[...] = jnp.zeros((BC, H * VW), jnp.float32)
        mscr[...] = jnp.full((BC, H), -jnp.inf, jnp.float32)

    neg = -jnp.inf
    bias = jnp.where(mask_ref[...] > 0.0, 0.0, neg)
    for h in range(H):
        qh = q_ref[:, h * DH:(h + 1) * DH]
        kh = k_ref[pl.ds(j * BN, BN), h * DH:(h + 1) * DH]
        s = lax.dot_general(qh, kh, (((1,), (1,)), ((), ())),
                            preferred_element_type=jnp.float32) + bias
        mo = mscr[:, h:h + 1]
        mn = jnp.maximum(mo, jnp.max(s, axis=1, keepdims=True))
        msafe = jnp.where(mn > neg, mn, 0.0)
        p = jnp.exp(s - msafe)
        alpha = jnp.where(mn > neg, jnp.exp(mo - mn), 0.0)
        vh = v_ref[pl.ds(j * BN, BN), VW * h:VW * (h + 1)]
        pv = lax.dot_general(p, vh, (((1,), (0,)), ((), ())),
                             preferred_element_type=jnp.float32)
        acc[:, VW * h:VW * (h + 1)] = acc[:, VW * h:VW * (h + 1)] * alpha + pv
        mscr[:, h:h + 1] = mn

    @pl.when(j == NJ - 1)
    def _():
        parts = [acc[:, VW * h:VW * h + DH] /
                 acc[:, VW * h + DH:VW * h + DH + 1] for h in range(H)]
        ctx = jnp.concatenate(parts, axis=1)
        ctxp = lax.dot_general(ctx, wo_ref[...], (((1,), (1,)), ((), ())),
                               preferred_element_type=jnp.float32) + bo_ref[...]
        has = acc[:, DH:DH + 1] > 0.0
        c2 = jnp.where(has, ctxp, emb_ref[...])
        h1 = lax.dot_general(c2, wl_ref[...], (((1,), (1,)), ((), ())),
                             preferred_element_type=jnp.float32) + bl_ref[...]
        mu = jnp.mean(h1, axis=1, keepdims=True)
        var = jnp.mean((h1 - mu) ** 2, axis=1, keepdims=True)
        hn = (h1 - mu) / jnp.sqrt(var + 1e-5) * lnw_ref[...] + lnb_ref[...]
        out_ref[...] = 0.5 * hn * (1.0 + lax.erf(hn * (2.0 ** -0.5)))


def _flash_piece(p, q, k, v, maskp, emb_pad, wo, bo, wl, bl, lnw, lnb):
    i0 = PIECE_LO[p] // BC
    ni_p = PIECE_ROWS[p] // BC

    def cmap(i, j, i0=i0):
        return (i + i0, 0)

    return pl.pallas_call(
        _flash_body,
        grid=(ni_p, NJ),
        in_specs=[
            pl.BlockSpec((BC, D), cmap),
            pl.BlockSpec((N_PAD, D), lambda i, j: (0, 0)),
            pl.BlockSpec((N_PAD, H * VW), lambda i, j: (0, 0)),
            pl.BlockSpec((BC, BN), lambda i, j: (i, j)),
            pl.BlockSpec((BC, D), cmap),
            pl.BlockSpec((D, D), lambda i, j: (0, 0)),
            pl.BlockSpec((1, D), lambda i, j: (0, 0)),
            pl.BlockSpec((D, D), lambda i, j: (0, 0)),
            pl.BlockSpec((1, D), lambda i, j: (0, 0)),
            pl.BlockSpec((1, D), lambda i, j: (0, 0)),
            pl.BlockSpec((1, D), lambda i, j: (0, 0)),
        ],
        out_specs=pl.BlockSpec((BC, D), lambda i, j: (i, 0)),
        out_shape=jax.ShapeDtypeStruct((PIECE_ROWS[p], D), jnp.float32),
        scratch_shapes=[
            pltpu.VMEM((BC, H * VW), jnp.float32),
            pltpu.VMEM((BC, H), jnp.float32),
        ],
        compiler_params=pltpu.CompilerParams(
            dimension_semantics=("arbitrary", "arbitrary")),
    )(q, k, v, maskp, emb_pad, wo, bo, wl, bl, lnw, lnb)


def kernel(embeddings, edge_index, in_proj_w, in_proj_b, out_proj_w,
           out_proj_b, lin_w, lin_b, ln_w, ln_b):
    emb_pad = jnp.zeros((N_PAD, D), jnp.float32).at[:N].set(embeddings)
    src = edge_index[0].astype(jnp.int32)
    dst = edge_index[1].astype(jnp.int32)

    q, k, v = _proj(emb_pad, in_proj_w, in_proj_b.reshape(1, 3 * D))

    masks = []
    for p in range(NPIECE):
        rows = PIECE_ROWS[p]
        mref = jax.new_ref(jnp.zeros(((rows + 8) * N_PAD,), jnp.float32))
        _SC_SCATTERS[p](src, dst, mref)
        masks.append(mref[...].reshape(rows + 8, N_PAD))

    outs = []
    for p in range(NPIECE):
        outs.append(_flash_piece(
            p, q, k, v, masks[p], emb_pad,
            out_proj_w, out_proj_b.reshape(1, D),
            lin_w, lin_b.reshape(1, D),
            ln_w.reshape(1, D), ln_b.reshape(1, D)))
    return jnp.concatenate(outs, axis=0)[:N]


# speedup vs baseline: 1.0106x; 1.0010x over previous
import jax
import jax.numpy as jnp
from jax import lax
from jax.experimental import pallas as pl
from jax.experimental.pallas import tpu as pltpu
from jax.experimental.pallas import tpu_sc as plsc

N = 10000
NE = 320000
D = 128
H = 8
DH = D // H
SCALE = 1.0 / (DH ** 0.5)

N_PAD = 10240
BC = 256
BN = 1024
NJ = N_PAD // BN

NC = 2
NS = 16
NW = NC * NS
E_PER_W = NE // NW
EBATCH = 2000
NBATCH = E_PER_W // EBATCH
VPB = EBATCH // 16
CHUNKS = 160

PIECE_ROWS = [1280] * 8
PIECE_LO = [sum(PIECE_ROWS[:p]) for p in range(len(PIECE_ROWS))]
NPIECE = len(PIECE_ROWS)


def _make_sc_body(lo, rows):
    hi = lo + rows
    dump = rows * N_PAD

    def body(src_hbm, dst_hbm, mask_hbm, idx1d, idx2d, ones_v, sbuf, dbuf,
             ssem):
        cid = lax.axis_index("c")
        sid = lax.axis_index("s")
        wid = cid * NS + sid
        lanes = lax.iota(jnp.int32, 16)
        for g in range(8):
            ones_v[pl.ds(g * 16, 16)] = jnp.ones((16,), jnp.float32)

        ebase = wid * E_PER_W

        def batch_body(b, ptr):
            pltpu.sync_copy(src_hbm.at[pl.ds(ebase + b * EBATCH, EBATCH)],
                            sbuf)
            pltpu.sync_copy(dst_hbm.at[pl.ds(ebase + b * EBATCH, EBATCH)],
                            dbuf)

            def vec(i, ptr):
                s16 = sbuf[pl.ds(i * 16, 16)]
                d16 = dbuf[pl.ds(i * 16, 16)]
                nsl = s16 != d16
                in1 = nsl & (s16 >= lo) & (s16 < hi)
                in2 = nsl & (d16 >= lo) & (d16 < hi)
                f1 = (s16 - lo) * N_PAD + d16
                f2 = (d16 - lo) * N_PAD + s16
                cs1 = plsc.cumsum(in1.astype(jnp.int32))
                plsc.store_scatter(idx1d, [ptr + cs1 - 1], f1, mask=in1)
                ptr = ptr + cs1[15]
                cs2 = plsc.cumsum(in2.astype(jnp.int32))
                plsc.store_scatter(idx1d, [ptr + cs2 - 1], f2, mask=in2)
                ptr = ptr + cs2[15]
                return ptr
            return lax.fori_loop(0, VPB, vec, ptr)
        ptr = lax.fori_loop(0, NBATCH, batch_body, jnp.int32(0))

        def pad(t, c):
            idx1d[pl.ds(ptr + t * 16, 16)] = dump + lanes
            return c
        lax.fori_loop(0, 8, pad, 0)
        nchunks = (ptr + 127) // 128

        def cp(i, c):
            idx2d[i // 8, pl.ds((i % 8) * 16, 16)] = idx1d[pl.ds(i * 16, 16)]
            return c
        lax.fori_loop(0, nchunks * 8, cp, 0)

        def fire(t, c):
            pltpu.async_copy(ones_v, mask_hbm.at[idx2d.at[t]], ssem)
            return c
        lax.fori_loop(0, nchunks, fire, 0)

        def drain(t, c):
            pltpu.make_async_copy(ones_v, mask_hbm.at[idx2d.at[0]],
                                  ssem).wait()
            return c
        lax.fori_loop(0, nchunks, drain, 0)
    return body


def _make_sc_scatter(lo, rows):
    return pl.kernel(
        _make_sc_body(lo, rows),
        out_type=(),
        compiler_params=pltpu.CompilerParams(needs_layout_passes=False),
        mesh=plsc.VectorSubcoreMesh(core_axis_name="c", subcore_axis_name="s"),
        scratch_types=[
            pltpu.VMEM((20608,), jnp.int32),
            pltpu.VMEM((CHUNKS, 128), jnp.int32),
            pltpu.VMEM((128,), jnp.float32),
            pltpu.VMEM((EBATCH,), jnp.int32),
            pltpu.VMEM((EBATCH,), jnp.int32),
            pltpu.SemaphoreType.DMA,
        ],
    )


_SC_SCATTERS = [_make_sc_scatter(PIECE_LO[p], PIECE_ROWS[p])
                for p in range(NPIECE)]


VW = 32


def _proj_body(emb_ref, w_ref, b_ref, q_ref, k_ref, v_ref):
    bp = emb_ref.shape[0]
    y = lax.dot_general(emb_ref[...], w_ref[...], (((1,), (1,)), ((), ())),
                        preferred_element_type=jnp.float32) + b_ref[...]
    q_ref[...] = y[:, :D] * SCALE
    k_ref[...] = y[:, D:2 * D]
    col = lax.broadcasted_iota(jnp.int32, (bp, VW - DH), 1)
    tail = jnp.where(col == 0, 1.0, 0.0)
    for h in range(H):
        v_ref[:, VW * h:VW * h + DH] = y[:, 2 * D + DH * h:2 * D + DH * (h + 1)]
        v_ref[:, VW * h + DH:VW * (h + 1)] = tail


def _proj(emb_pad, w_all, b_all):
    bp = 512
    return pl.pallas_call(
        _proj_body,
        grid=(N_PAD // bp,),
        in_specs=[
            pl.BlockSpec((bp, D), lambda i: (i, 0)),
            pl.BlockSpec((3 * D, D), lambda i: (0, 0)),
            pl.BlockSpec((1, 3 * D), lambda i: (0, 0)),
        ],
        out_specs=[
            pl.BlockSpec((bp, D), lambda i: (i, 0)),
            pl.BlockSpec((bp, D), lambda i: (i, 0)),
            pl.BlockSpec((bp, H * VW), lambda i: (i, 0)),
        ],
        out_shape=[jax.ShapeDtypeStruct((N_PAD, D), jnp.float32),
                   jax.ShapeDtypeStruct((N_PAD, D), jnp.float32),
                   jax.ShapeDtypeStruct((N_PAD, H * VW), jnp.float32)],
    )(emb_pad, w_all, b_all)


def _flash_body(q_ref, k_ref, v_ref, mask_ref, emb_ref, wo_ref, bo_ref,
                wl_ref, bl_ref, lnw_ref, lnb_ref, out_ref,
                acc, mscr):
    j = pl.program_id(1)

    @pl.when(j == 0)
    def _():
        acc[...] = jnp.zeros((BC, H * VW), jnp.float32)
        mscr[...] = jnp.full((BC, H), -jnp.inf, jnp.float32)

    neg = -jnp.inf
    bias = jnp.where(mask_ref[...] > 0.0, 0.0, neg)
    for h in range(H):
        qh = q_ref[:, h * DH:(h + 1) * DH]
        kh = k_ref[pl.ds(j * BN, BN), h * DH:(h + 1) * DH]
        s = lax.dot_general(qh, kh, (((1,), (1,)), ((), ())),
                            preferred_element_type=jnp.float32) + bias
        mo = mscr[:, h:h + 1]
        mn = jnp.maximum(mo, jnp.max(s, axis=1, keepdims=True))
        msafe = jnp.where(mn > neg, mn, 0.0)
        p = jnp.exp(s - msafe)
        alpha = jnp.where(mn > neg, jnp.exp(mo - mn), 0.0)
        vh = v_ref[pl.ds(j * BN, BN), VW * h:VW * (h + 1)]
        pv = lax.dot_general(p, vh, (((1,), (0,)), ((), ())),
                             preferred_element_type=jnp.float32)
        acc[:, VW * h:VW * (h + 1)] = acc[:, VW * h:VW * (h + 1)] * alpha + pv
        mscr[:, h:h + 1] = mn

    @pl.when(j == NJ - 1)
    def _():
        parts = [acc[:, VW * h:VW * h + DH] /
                 acc[:, VW * h + DH:VW * h + DH + 1] for h in range(H)]
        ctx = jnp.concatenate(parts, axis=1)
        ctxp = lax.dot_general(ctx, wo_ref[...], (((1,), (1,)), ((), ())),
                               preferred_element_type=jnp.float32) + bo_ref[...]
        has = acc[:, DH:DH + 1] > 0.0
        c2 = jnp.where(has, ctxp, emb_ref[...])
        h1 = lax.dot_general(c2, wl_ref[...], (((1,), (1,)), ((), ())),
                             preferred_element_type=jnp.float32) + bl_ref[...]
        mu = jnp.mean(h1, axis=1, keepdims=True)
        var = jnp.mean((h1 - mu) ** 2, axis=1, keepdims=True)
        hn = (h1 - mu) / jnp.sqrt(var + 1e-5) * lnw_ref[...] + lnb_ref[...]
        out_ref[...] = 0.5 * hn * (1.0 + lax.erf(hn * (2.0 ** -0.5)))


def _flash_piece(p, q, k, v, maskp, emb_pad, wo, bo, wl, bl, lnw, lnb):
    i0 = PIECE_LO[p] // BC
    ni_p = PIECE_ROWS[p] // BC

    def cmap(i, j, i0=i0):
        return (i + i0, 0)

    return pl.pallas_call(
        _flash_body,
        grid=(ni_p, NJ),
        in_specs=[
            pl.BlockSpec((BC, D), cmap),
            pl.BlockSpec((N_PAD, D), lambda i, j: (0, 0)),
            pl.BlockSpec((N_PAD, H * VW), lambda i, j: (0, 0)),
            pl.BlockSpec((BC, BN), lambda i, j: (i, j)),
            pl.BlockSpec((BC, D), cmap),
            pl.BlockSpec((D, D), lambda i, j: (0, 0)),
            pl.BlockSpec((1, D), lambda i, j: (0, 0)),
            pl.BlockSpec((D, D), lambda i, j: (0, 0)),
            pl.BlockSpec((1, D), lambda i, j: (0, 0)),
            pl.BlockSpec((1, D), lambda i, j: (0, 0)),
            pl.BlockSpec((1, D), lambda i, j: (0, 0)),
        ],
        out_specs=pl.BlockSpec((BC, D), lambda i, j: (i, 0)),
        out_shape=jax.ShapeDtypeStruct((PIECE_ROWS[p], D), jnp.float32),
        scratch_shapes=[
            pltpu.VMEM((BC, H * VW), jnp.float32),
            pltpu.VMEM((BC, H), jnp.float32),
        ],
        compiler_params=pltpu.CompilerParams(
            dimension_semantics=("arbitrary", "arbitrary")),
    )(q, k, v, maskp, emb_pad, wo, bo, wl, bl, lnw, lnb)


def kernel(embeddings, edge_index, in_proj_w, in_proj_b, out_proj_w,
           out_proj_b, lin_w, lin_b, ln_w, ln_b):
    emb_pad = jnp.zeros((N_PAD, D), jnp.float32).at[:N].set(embeddings)
    src = edge_index[0].astype(jnp.int32)
    dst = edge_index[1].astype(jnp.int32)

    q, k, v = _proj(emb_pad, in_proj_w, in_proj_b.reshape(1, 3 * D))

    masks = []
    for p in range(NPIECE):
        rows = PIECE_ROWS[p]
        mref = jax.new_ref(jnp.zeros(((rows + 8) * N_PAD,), jnp.float32))
        _SC_SCATTERS[p](src, dst, mref)
        masks.append(mref[...].reshape(rows + 8, N_PAD))

    outs = []
    for p in range(NPIECE):
        outs.append(_flash_piece(
            p, q, k, v, masks[p], emb_pad,
            out_proj_w, out_proj_b.reshape(1, D),
            lin_w, lin_b.reshape(1, D),
            ln_w.reshape(1, D), ln_b.reshape(1, D)))
    return jnp.concatenate(outs, axis=0)[:N]
